# baseline (speedup 1.0000x reference)
import functools

import jax
import jax.numpy as jnp
from jax import lax
from jax.experimental import pallas as pl
from jax.experimental.pallas import tpu as pltpu
from jax.experimental.pallas import tpu_sc as plsc

D = 64
NC, NS = 2, 16
NW = NC * NS


@functools.partial(jax.jit, static_argnums=(3, 4))
def _lookup_sum(ixy, wx, wy, n, c):
    per_w = n // NW
    n_chunks = per_w // c
    assert n_chunks % 2 == 0
    mesh = plsc.VectorSubcoreMesh(core_axis_name="c", subcore_axis_name="s")

    def body(ixy_hbm, wx_hbm, wy_hbm, out_hbm,
             ixy_v, ix0, ix1, iy0, iy1, ba0, ba1, bb0, bb1,
             ga0, ga1, gb0, gb1, wb0, wb1):
        wid = lax.axis_index("s") * NC + lax.axis_index("c")
        base = wid * per_w
        idxx = (ix0, ix1)
        idxy = (iy0, iy1)
        bufa = (ba0, ba1)
        bufb = (bb0, bb1)
        ga = (ga0, ga1)
        gb = (gb0, gb1)
        wb = (wb0, wb1)

        pltpu.sync_copy(ixy_hbm.at[pl.ds(2 * base, 2 * per_w)], ixy_v)

        lanes = lax.iota(jnp.int32, 16)
        ge = (2 * lanes) & 15
        go = (2 * lanes + 1) & 15
        lo = lanes < 8

        gdn = lax.GatherDimensionNumbers(
            offset_dims=(), collapsed_slice_dims=(0,), start_index_map=(0,))

        def lane_take(v, idx):
            return lax.gather(
                v, idx[:, None], gdn, slice_sizes=(1,),
                mode=lax.GatherScatterMode.PROMISE_IN_BOUNDS)

        def stage_and_fire(k, b):
            for j in range(c // 16):
                p = 2 * c * k + 32 * j
                va = ixy_v[pl.ds(p, 16)]
                vb = ixy_v[pl.ds(p + 16, 16)]
                vx = jnp.where(lo, lane_take(va, ge),
                               lane_take(vb, ge))
                vy = jnp.where(lo, lane_take(va, go),
                               lane_take(vb, go))
                idxx[b][pl.ds(j * 16, 16)] = vx
                idxy[b][pl.ds(j * 16, 16)] = vy
            pltpu.async_copy(wx_hbm.at[idxx[b]], bufa[b], ga[b])
            pltpu.async_copy(wy_hbm.at[idxy[b]], bufb[b], gb[b])

        def wait_gathers(b):
            pltpu.make_async_copy(wx_hbm.at[idxx[b]], bufa[b], ga[b]).wait()
            pltpu.make_async_copy(wy_hbm.at[idxy[b]], bufb[b], gb[b]).wait()

        def wait_wb(k, b):
            pltpu.make_async_copy(
                bufa[b], out_hbm.at[pl.ds(base + k * c, c)], wb[b]).wait()

        stage_and_fire(0, 0)

        def pair(k2, carry):
            for b in (0, 1):
                k = 2 * k2 + b
                b1 = 1 - b

                @pl.when(k >= 1)
                def _():
                    wait_wb(k - 1, b1)

                @pl.when(k + 1 < n_chunks)
                def _():
                    stage_and_fire(k + 1, b1)

                wait_gathers(b)

                def add_row(i, carry2):
                    for j in range(D // 16):
                        s = pl.ds(j * 16, 16)
                        bufa[b][i, s] = bufa[b][i, s] + bufb[b][i, s]
                    return carry2

                lax.fori_loop(0, c, add_row, 0, unroll=4)
                pltpu.async_copy(
                    bufa[b], out_hbm.at[pl.ds(base + k * c, c)], wb[b])
            return carry

        lax.fori_loop(0, n_chunks // 2, pair, 0)
        wait_wb(n_chunks - 1, 1)

    return pl.kernel(
        body,
        out_type=jax.ShapeDtypeStruct((n, D), jnp.float32),
        mesh=mesh,
        compiler_params=pltpu.CompilerParams(use_tc_tiling_on_sc=False),
        scratch_types=[
            pltpu.VMEM((2 * per_w,), jnp.int32),
            pltpu.VMEM((c,), jnp.int32),
            pltpu.VMEM((c,), jnp.int32),
            pltpu.VMEM((c,), jnp.int32),
            pltpu.VMEM((c,), jnp.int32),
            pltpu.VMEM((c, D), jnp.float32),
            pltpu.VMEM((c, D), jnp.float32),
            pltpu.VMEM((c, D), jnp.float32),
            pltpu.VMEM((c, D), jnp.float32),
            pltpu.SemaphoreType.DMA,
            pltpu.SemaphoreType.DMA,
            pltpu.SemaphoreType.DMA,
            pltpu.SemaphoreType.DMA,
            pltpu.SemaphoreType.DMA,
            pltpu.SemaphoreType.DMA,
        ],
    )(ixy, wx, wy)


def kernel(x_coord, Wx, Wy):
    b, l, _ = x_coord.shape
    n = b * l
    ixy = x_coord.reshape(2 * n)
    out = _lookup_sum(ixy, Wx, Wy, n, 128)
    return out.reshape(b, l, D)


# speedup vs baseline: 2.5907x; 2.5907x over previous
import functools

import jax
import jax.numpy as jnp
from jax import lax
from jax.experimental import pallas as pl
from jax.experimental.pallas import tpu as pltpu
from jax.experimental.pallas import tpu_sc as plsc

D = 64
NC, NS = 2, 16
NW = NC * NS


@functools.partial(jax.jit, static_argnums=(4, 5))
def _lookup_sum(ix, iy, wx, wy, n, c):
    per_w = n // NW
    n_chunks = per_w // c
    assert n_chunks % 2 == 0
    mesh = plsc.VectorSubcoreMesh(core_axis_name="c", subcore_axis_name="s")

    def body(ix_hbm, iy_hbm, wx_hbm, wy_hbm, out_hbm,
             idxx_v, idxy_v, ba0, ba1, bb0, bb1,
             ga0, ga1, gb0, gb1, wb0, wb1):
        wid = lax.axis_index("s") * NC + lax.axis_index("c")
        base = wid * per_w
        bufa = (ba0, ba1)
        bufb = (bb0, bb1)
        ga = (ga0, ga1)
        gb = (gb0, gb1)
        wb = (wb0, wb1)

        pltpu.sync_copy(ix_hbm.at[pl.ds(base, per_w)], idxx_v)
        pltpu.sync_copy(iy_hbm.at[pl.ds(base, per_w)], idxy_v)

        def fire_gathers(k, b):
            s = pl.ds(k * c, c)
            pltpu.async_copy(wx_hbm.at[idxx_v.at[s]], bufa[b], ga[b])
            pltpu.async_copy(wy_hbm.at[idxy_v.at[s]], bufb[b], gb[b])

        def wait_gathers(k, b):
            s = pl.ds(k * c, c)
            pltpu.make_async_copy(wx_hbm.at[idxx_v.at[s]], bufa[b], ga[b]).wait()
            pltpu.make_async_copy(wy_hbm.at[idxy_v.at[s]], bufb[b], gb[b]).wait()

        def wait_wb(k, b):
            pltpu.make_async_copy(
                bufa[b], out_hbm.at[pl.ds(base + k * c, c)], wb[b]).wait()

        fire_gathers(0, 0)

        def pair(k2, carry):
            for b in (0, 1):
                k = 2 * k2 + b
                b1 = 1 - b

                @pl.when(k >= 1)
                def _():
                    wait_wb(k - 1, b1)

                @pl.when(k + 1 < n_chunks)
                def _():
                    fire_gathers(k + 1, b1)

                wait_gathers(k, b)

                @plsc.parallel_loop(0, c, unroll=8)
                def add_row(i):
                    for j in range(D // 16):
                        s = pl.ds(j * 16, 16)
                        bufa[b][i, s] = bufa[b][i, s] + bufb[b][i, s]

                pltpu.async_copy(
                    bufa[b], out_hbm.at[pl.ds(base + k * c, c)], wb[b])
            return carry

        lax.fori_loop(0, n_chunks // 2, pair, 0)
        wait_wb(n_chunks - 1, 1)

    return pl.kernel(
        body,
        out_type=jax.ShapeDtypeStruct((n, D), jnp.float32),
        mesh=mesh,
        compiler_params=pltpu.CompilerParams(use_tc_tiling_on_sc=False),
        scratch_types=[
            pltpu.VMEM((per_w,), jnp.int32),
            pltpu.VMEM((per_w,), jnp.int32),
            pltpu.VMEM((c, D), jnp.float32),
            pltpu.VMEM((c, D), jnp.float32),
            pltpu.VMEM((c, D), jnp.float32),
            pltpu.VMEM((c, D), jnp.float32),
            pltpu.SemaphoreType.DMA,
            pltpu.SemaphoreType.DMA,
            pltpu.SemaphoreType.DMA,
            pltpu.SemaphoreType.DMA,
            pltpu.SemaphoreType.DMA,
            pltpu.SemaphoreType.DMA,
        ],
    )(ix, iy, wx, wy)


def kernel(x_coord, Wx, Wy):
    b, l, _ = x_coord.shape
    n = b * l
    ix = x_coord[..., 0].reshape(n)
    iy = x_coord[..., 1].reshape(n)
    out = _lookup_sum(ix, iy, Wx, Wy, n, 256)
    return out.reshape(b, l, D)
